# baseline (speedup 1.0000x reference)
import functools

import jax
import jax.numpy as jnp
from jax import lax
from jax.experimental import pallas as pl
from jax.experimental.pallas import tpu as pltpu
from jax.experimental.pallas import tpu_sc as plsc

N = 10000
E = 160000
D = 256
NG = 64
NC = 2
NS = 16
LANES = 16

EPAD = 161792
EPT_DEG = EPAD // (NC * NS)
TROWS = 320
ACCR = TROWS + 8
SENT = TROWS
SEG = 2048
NSEG = EPAD // SEG
GCH = 64
NDEG = N + LANES

RB = 1000
NBLK = N // RB


def _deg_body(dst_hbm, out_hbm, dstv, degv):
    c = lax.axis_index("c")
    s = lax.axis_index("s")
    wid = c * NS + s
    pltpu.sync_copy(dst_hbm.at[pl.ds(wid * EPT_DEG, EPT_DEG)], dstv)
    zeros = jnp.zeros((LANES,), jnp.float32)

    def zbody(i, carry):
        degv[pl.ds(i * LANES, LANES)] = zeros
        return carry

    lax.fori_loop(0, NDEG // LANES, zbody, 0)
    ones = jnp.ones((LANES,), jnp.float32)

    def abody(i, carry):
        idx = dstv[pl.ds(i * LANES, LANES)]
        plsc.addupdate_scatter(degv, [idx], ones)
        return carry

    lax.fori_loop(0, EPT_DEG // LANES, abody, 0)
    pltpu.sync_copy(degv, out_hbm.at[wid])


def _agg_body(hs_hbm, src_hbm, dst_hbm, out_hbm,
              srcv, dstv, csrc, cdst, rowb0, rowb1, acc, sem0, sem1):
    c = lax.axis_index("c")
    s = lax.axis_index("s")
    w = s * NC + c
    base = w * TROWS
    rsize = jnp.minimum(TROWS, N - base)

    zeros = jnp.zeros((LANES,), jnp.float32)

    def zrow(r, carry):
        def zcol(j, carry2):
            acc[r, pl.ds(j * LANES, LANES)] = zeros
            return carry2
        return lax.fori_loop(0, D // LANES, zcol, carry)

    lax.fori_loop(0, ACCR, zrow, 0)

    sent_s = jnp.zeros((LANES,), jnp.int32)
    sent_d = jnp.full((LANES,), SENT, jnp.int32)
    rowbs = (rowb0, rowb1)
    sems = (sem0, sem1)

    def issue(rb, sem, kk16):
        csv = csrc[pl.ds(kk16, LANES)]
        for g in range(LANES):
            soff = pl.multiple_of(csv[g] * D, 8)
            pltpu.async_copy(hs_hbm.at[pl.ds(soff, D)],
                             rb.at[pl.ds(g * D, D)], sem)

    def drain(rb, sem):
        pltpu.make_async_copy(hs_hbm.at[pl.ds(0, LANES * D)], rb, sem).wait()

    def accum(rb, kk16):
        dloc = cdst[pl.ds(kk16, LANES)]
        for g in range(LANES):
            d = dloc[g]
            vals = [rb[pl.ds(g * D + j * LANES, LANES)] for j in range(D // LANES)]
            for j in range(D // LANES):
                plsc.addupdate(acc.at[d, pl.ds(j * LANES, LANES)], vals[j])

    def seg_body(g, carry):
        pltpu.sync_copy(src_hbm.at[pl.ds(g * SEG, SEG)], srcv)
        pltpu.sync_copy(dst_hbm.at[pl.ds(g * SEG, SEG)], dstv)

        def fbody(i, n):
            dv = dstv[pl.ds(i * LANES, LANES)]
            sv = srcv[pl.ds(i * LANES, LANES)]
            dloc = dv - base
            m = (dloc >= 0) & (dloc < rsize)
            plsc.store_compressed(csrc.at[pl.ds(n, LANES)], sv, mask=m)
            plsc.store_compressed(cdst.at[pl.ds(n, LANES)], dloc, mask=m)
            return n + plsc.all_reduce_population_count(m)[0]

        n = lax.fori_loop(0, SEG // LANES, fbody, jnp.int32(0))

        csrc[pl.ds(n, LANES)] = sent_s
        cdst[pl.ds(n, LANES)] = sent_d
        ngr = (n + LANES - 1) // LANES

        def gbody(k, carry2):
            kk16 = k * LANES
            issue(rowb0, sem0, kk16)
            drain(rowb0, sem0)
            accum(rowb0, kk16)
            return carry2

        lax.fori_loop(0, ngr, gbody, 0)
        return carry

    lax.fori_loop(0, NSEG, seg_body, 0)

    hoff = pl.multiple_of(base, 8)

    @pl.when(rsize == TROWS)
    def _():
        pltpu.sync_copy(acc.at[pl.ds(0, TROWS)], out_hbm.at[pl.ds(hoff, TROWS)])

    @pl.when(rsize < TROWS)
    def _():
        pltpu.sync_copy(acc.at[pl.ds(0, N - (NC * NS - 1) * TROWS)],
                        out_hbm.at[pl.ds(hoff, N - (NC * NS - 1) * TROWS)])


def _deg_call(dstp):
    return pl.kernel(
        _deg_body,
        out_type=jax.ShapeDtypeStruct((NC * NS, NDEG), jnp.float32),
        mesh=plsc.VectorSubcoreMesh(core_axis_name="c", subcore_axis_name="s"),
        compiler_params=pltpu.CompilerParams(needs_layout_passes=False),
        scratch_types=[
            pltpu.VMEM((EPT_DEG,), jnp.int32),
            pltpu.VMEM((NDEG,), jnp.float32),
        ],
    )(dstp)


def _agg_call(hs, srcp, dstp):
    return pl.kernel(
        _agg_body,
        out_type=jax.ShapeDtypeStruct((N, D), jnp.float32),
        mesh=plsc.VectorSubcoreMesh(core_axis_name="c", subcore_axis_name="s"),
        compiler_params=pltpu.CompilerParams(needs_layout_passes=False),
        scratch_types=[
            pltpu.VMEM((SEG,), jnp.int32),
            pltpu.VMEM((SEG,), jnp.int32),
            pltpu.VMEM((SEG + GCH,), jnp.int32),
            pltpu.VMEM((SEG + GCH,), jnp.int32),
            pltpu.VMEM((LANES * D,), jnp.float32),
            pltpu.VMEM((LANES * D,), jnp.float32),
            pltpu.VMEM((ACCR, D), jnp.float32),
            pltpu.SemaphoreType.DMA,
            pltpu.SemaphoreType.DMA,
        ],
    )(hs, srcp, dstp)


def _prep_body(x_ref, w_ref, degp_ref, out_ref):
    dinv = lax.rsqrt(jnp.sum(degp_ref[...], axis=1) + 1.0)
    out_ref[...] = jnp.dot(x_ref[...], w_ref[...],
                           preferred_element_type=jnp.float32) * dinv[:, None]


def _prep_call(x, W1, degp):
    return pl.pallas_call(
        _prep_body,
        grid=(NBLK,),
        in_specs=[
            pl.BlockSpec((RB, D), lambda i: (i, 0)),
            pl.BlockSpec((D, D), lambda i: (0, 0)),
            pl.BlockSpec((RB, NC * NS), lambda i: (i, 0)),
        ],
        out_specs=pl.BlockSpec((RB, D), lambda i: (i, 0)),
        out_shape=jax.ShapeDtypeStruct((N, D), jnp.float32),
    )(x, W1, degp)


def _mid_body(agg_ref, hs_ref, degp_ref, w_ref, b_ref, out_ref):
    dinv = lax.rsqrt(jnp.sum(degp_ref[...], axis=1) + 1.0)
    h1 = jnp.maximum((agg_ref[...] + hs_ref[...]) * dinv[:, None] + b_ref[...], 0.0)
    out_ref[...] = jnp.dot(h1, w_ref[...],
                           preferred_element_type=jnp.float32) * dinv[:, None]


def _mid_call(agg1, hs1, degp, W2, b1):
    return pl.pallas_call(
        _mid_body,
        grid=(NBLK,),
        in_specs=[
            pl.BlockSpec((RB, D), lambda i: (i, 0)),
            pl.BlockSpec((RB, D), lambda i: (i, 0)),
            pl.BlockSpec((RB, NC * NS), lambda i: (i, 0)),
            pl.BlockSpec((D, D), lambda i: (0, 0)),
            pl.BlockSpec((D,), lambda i: (0,)),
        ],
        out_specs=pl.BlockSpec((RB, D), lambda i: (i, 0)),
        out_shape=jax.ShapeDtypeStruct((N, D), jnp.float32),
    )(agg1, hs1, degp, W2, b1)


def _final_body(agg_ref, hs_ref, degp_ref, b_ref, batch_ref, glob_ref,
                wh1_ref, bh1_ref, wh2_ref, bh2_ref, out_ref,
                pooled_acc, cnt_acc):
    i = pl.program_id(0)
    dinv = lax.rsqrt(jnp.sum(degp_ref[...], axis=1) + 1.0)
    h2 = (agg_ref[...] + hs_ref[...]) * dinv[:, None] + b_ref[...]
    gids = lax.broadcasted_iota(jnp.int32, (RB, NG), 1)
    m = (gids == batch_ref[...]).astype(jnp.float32)

    @pl.when(i == 0)
    def _():
        pooled_acc[...] = jnp.zeros_like(pooled_acc)
        cnt_acc[...] = jnp.zeros_like(cnt_acc)

    pooled_acc[...] += lax.dot_general(m, h2, (((0,), (0,)), ((), ())),
                                       preferred_element_type=jnp.float32)
    cnt_acc[...] += jnp.broadcast_to(jnp.sum(m, axis=0)[:, None],
                                     cnt_acc.shape)

    @pl.when(i == NBLK - 1)
    def _():
        cnt = cnt_acc[...][:, 0:1]
        pooled = pooled_acc[...] / jnp.maximum(cnt, 1.0)
        z1 = (jnp.dot(pooled, wh1_ref[0:D, :], preferred_element_type=jnp.float32)
              + jnp.dot(glob_ref[...], wh1_ref[D:D + 10, :],
                        preferred_element_type=jnp.float32)
              + bh1_ref[...])
        z1 = jnp.maximum(z1, 0.0)
        z2 = jnp.dot(z1, wh2_ref[...], preferred_element_type=jnp.float32) + bh2_ref[...]
        out_ref[...] = jax.nn.sigmoid(z2)


def _final_call(agg2, hs2, degp, b2, batch2d, glob_vecs, Wh1, bh1, Wh2, bh2):
    return pl.pallas_call(
        _final_body,
        grid=(NBLK,),
        in_specs=[
            pl.BlockSpec((RB, D), lambda i: (i, 0)),
            pl.BlockSpec((RB, D), lambda i: (i, 0)),
            pl.BlockSpec((RB, NC * NS), lambda i: (i, 0)),
            pl.BlockSpec((D,), lambda i: (0,)),
            pl.BlockSpec((RB, 1), lambda i: (i, 0)),
            pl.BlockSpec((NG, 10), lambda i: (0, 0)),
            pl.BlockSpec((D + 10, 10), lambda i: (0, 0)),
            pl.BlockSpec((1, 10), lambda i: (0, 0)),
            pl.BlockSpec((10, 1), lambda i: (0, 0)),
            pl.BlockSpec((1, 1), lambda i: (0, 0)),
        ],
        out_specs=pl.BlockSpec((NG, 1), lambda i: (0, 0)),
        out_shape=jax.ShapeDtypeStruct((NG, 1), jnp.float32),
        scratch_shapes=[
            pltpu.VMEM((NG, D), jnp.float32),
            pltpu.VMEM((NG, 128), jnp.float32),
        ],
    )(agg2, hs2, degp, b2, batch2d, glob_vecs, Wh1, bh1, Wh2, bh2)


def kernel(x, edge_index, edge_attr, batch_idx, glob_vecs,
           W1, b1, W2, b2, Wh1, bh1, Wh2, bh2):
    pad = EPAD - E
    srcp = jnp.concatenate([edge_index[0], jnp.zeros((pad,), jnp.int32)])
    dstp = jnp.concatenate([edge_index[1], jnp.full((pad,), N, jnp.int32)])

    degp = _deg_call(dstp)[:, :N].T
    hs1 = _prep_call(x, W1, degp)
    agg1 = _agg_call(hs1.reshape(N * D), srcp, dstp)
    hs2 = _mid_call(agg1, hs1, degp, W2, b1)
    agg2 = _agg_call(hs2.reshape(N * D), srcp, dstp)
    z = _final_call(agg2, hs2, degp, b2, batch_idx.reshape(N, 1),
                    glob_vecs, Wh1, bh1.reshape(1, 10), Wh2, bh2.reshape(1, 1))
    return z


# speedup vs baseline: 1.4614x; 1.4614x over previous
import functools

import jax
import jax.numpy as jnp
from jax import lax
from jax.experimental import pallas as pl
from jax.experimental.pallas import tpu as pltpu
from jax.experimental.pallas import tpu_sc as plsc

N = 10000
E = 160000
D = 256
NG = 64
NC = 2
NS = 16
LANES = 16

EPAD = 163840
SEG = 4096
NSEG = EPAD // SEG
EPALLOC = EPAD + SEG
EPT_DEG = EPAD // (NC * NS)
TROWS = 320
ACCR = TROWS + 8
SENT = TROWS
NDEG = N + LANES

RB = 1000
NBLK = N // RB


def _deg_body(dst_hbm, out_hbm, dstv, degv):
    c = lax.axis_index("c")
    s = lax.axis_index("s")
    wid = c * NS + s
    pltpu.sync_copy(dst_hbm.at[pl.ds(wid * EPT_DEG, EPT_DEG)], dstv)
    zeros = jnp.zeros((LANES,), jnp.float32)

    def zbody(i, carry):
        degv[pl.ds(i * LANES, LANES)] = zeros
        return carry

    lax.fori_loop(0, NDEG // LANES, zbody, 0)
    ones = jnp.ones((LANES,), jnp.float32)

    def abody(i, carry):
        idx = dstv[pl.ds(i * LANES, LANES)]
        plsc.addupdate_scatter(degv, [idx], ones)
        return carry

    lax.fori_loop(0, EPT_DEG // LANES, abody, 0)
    pltpu.sync_copy(degv, out_hbm.at[wid])


def _agg_body(hs_hbm, src_hbm, dst_hbm, out_hbm,
              srcv0, dstv0, srcv1, dstv1, csrc, cdst, rowb0, acc,
              sem0, semA, semB):
    c = lax.axis_index("c")
    s = lax.axis_index("s")
    w = s * NC + c
    base = w * TROWS
    rsize = jnp.minimum(TROWS, N - base)

    zeros = jnp.zeros((LANES,), jnp.float32)

    def zrow(r, carry):
        def zcol(j, carry2):
            acc[r, pl.ds(j * LANES, LANES)] = zeros
            return carry2
        return lax.fori_loop(0, D // LANES, zcol, carry)

    lax.fori_loop(0, ACCR, zrow, 0)

    sent_s = jnp.zeros((LANES,), jnp.int32)
    sent_d = jnp.full((LANES,), SENT, jnp.int32)

    def issue(rb, sem, kk16):
        csv = csrc[pl.ds(kk16, LANES)]
        for g in range(LANES):
            soff = pl.multiple_of(csv[g] * D, 8)
            pltpu.async_copy(hs_hbm.at[pl.ds(soff, D)],
                             rb.at[pl.ds(g * D, D)], sem)

    def drain(rb, sem):
        pltpu.make_async_copy(hs_hbm.at[pl.ds(0, LANES * D)], rb, sem).wait()

    def accum(rb, kk16):
        dloc = cdst[pl.ds(kk16, LANES)]
        for g in range(LANES):
            d = dloc[g]
            vals = [rb[pl.ds(g * D + j * LANES, LANES)] for j in range(D // LANES)]
            for j in range(D // LANES):
                plsc.addupdate(acc.at[d, pl.ds(j * LANES, LANES)], vals[j])

    def seg_work(sv_ref, dv_ref):
        def fbody(i, n):
            dv = dv_ref[pl.ds(i * LANES, LANES)]
            sv = sv_ref[pl.ds(i * LANES, LANES)]
            dloc = dv - base
            m = (dloc >= 0) & (dloc < rsize)
            plsc.store_compressed(csrc.at[pl.ds(n, LANES)], sv, mask=m)
            plsc.store_compressed(cdst.at[pl.ds(n, LANES)], dloc, mask=m)
            return n + plsc.all_reduce_population_count(m)[0]

        n = lax.fori_loop(0, SEG // LANES, fbody, jnp.int32(0))

        csrc[pl.ds(n, LANES)] = sent_s
        cdst[pl.ds(n, LANES)] = sent_d
        ngr = (n + LANES - 1) // LANES

        def gbody(k, carry2):
            kk16 = k * LANES
            issue(rowb0, sem0, kk16)
            drain(rowb0, sem0)
            accum(rowb0, kk16)
            return carry2

        lax.fori_loop(0, ngr, gbody, 0)

    def load_seg(g, sv_ref, dv_ref, sem):
        pltpu.async_copy(src_hbm.at[pl.ds(g * SEG, SEG)], sv_ref, sem)
        pltpu.async_copy(dst_hbm.at[pl.ds(g * SEG, SEG)], dv_ref, sem)

    def wait_seg(sv_ref, dv_ref, sem):
        pltpu.make_async_copy(src_hbm.at[pl.ds(0, SEG)], sv_ref, sem).wait()
        pltpu.make_async_copy(dst_hbm.at[pl.ds(0, SEG)], dv_ref, sem).wait()

    load_seg(0, srcv0, dstv0, semA)

    def pair_body(k, carry):
        g0 = 2 * k
        wait_seg(srcv0, dstv0, semA)
        load_seg(g0 + 1, srcv1, dstv1, semB)
        seg_work(srcv0, dstv0)
        wait_seg(srcv1, dstv1, semB)
        load_seg(g0 + 2, srcv0, dstv0, semA)
        seg_work(srcv1, dstv1)
        return carry

    lax.fori_loop(0, NSEG // 2, pair_body, 0)
    wait_seg(srcv0, dstv0, semA)

    hoff = pl.multiple_of(base, 8)

    @pl.when(rsize == TROWS)
    def _():
        pltpu.sync_copy(acc.at[pl.ds(0, TROWS)], out_hbm.at[pl.ds(hoff, TROWS)])

    @pl.when(rsize < TROWS)
    def _():
        pltpu.sync_copy(acc.at[pl.ds(0, N - (NC * NS - 1) * TROWS)],
                        out_hbm.at[pl.ds(hoff, N - (NC * NS - 1) * TROWS)])


def _deg_call(dstp):
    return pl.kernel(
        _deg_body,
        out_type=jax.ShapeDtypeStruct((NC * NS, NDEG), jnp.float32),
        mesh=plsc.VectorSubcoreMesh(core_axis_name="c", subcore_axis_name="s"),
        compiler_params=pltpu.CompilerParams(needs_layout_passes=False),
        scratch_types=[
            pltpu.VMEM((EPT_DEG,), jnp.int32),
            pltpu.VMEM((NDEG,), jnp.float32),
        ],
    )(dstp)


def _agg_call(hs, srcp, dstp):
    return pl.kernel(
        _agg_body,
        out_type=jax.ShapeDtypeStruct((N, D), jnp.float32),
        mesh=plsc.VectorSubcoreMesh(core_axis_name="c", subcore_axis_name="s"),
        compiler_params=pltpu.CompilerParams(needs_layout_passes=False),
        scratch_types=[
            pltpu.VMEM((SEG,), jnp.int32),
            pltpu.VMEM((SEG,), jnp.int32),
            pltpu.VMEM((SEG,), jnp.int32),
            pltpu.VMEM((SEG,), jnp.int32),
            pltpu.VMEM((SEG + LANES,), jnp.int32),
            pltpu.VMEM((SEG + LANES,), jnp.int32),
            pltpu.VMEM((LANES * D,), jnp.float32),
            pltpu.VMEM((ACCR, D), jnp.float32),
            pltpu.SemaphoreType.DMA,
            pltpu.SemaphoreType.DMA,
            pltpu.SemaphoreType.DMA,
        ],
    )(hs, srcp, dstp)


def _prep_body(x_ref, w_ref, degp_ref, out_ref):
    dinv = lax.rsqrt(jnp.sum(degp_ref[...], axis=1) + 1.0)
    out_ref[...] = jnp.dot(x_ref[...], w_ref[...],
                           preferred_element_type=jnp.float32) * dinv[:, None]


def _prep_call(x, W1, degp):
    return pl.pallas_call(
        _prep_body,
        grid=(NBLK,),
        in_specs=[
            pl.BlockSpec((RB, D), lambda i: (i, 0)),
            pl.BlockSpec((D, D), lambda i: (0, 0)),
            pl.BlockSpec((RB, NC * NS), lambda i: (i, 0)),
        ],
        out_specs=pl.BlockSpec((RB, D), lambda i: (i, 0)),
        out_shape=jax.ShapeDtypeStruct((N, D), jnp.float32),
    )(x, W1, degp)


def _mid_body(agg_ref, hs_ref, degp_ref, w_ref, b_ref, out_ref):
    dinv = lax.rsqrt(jnp.sum(degp_ref[...], axis=1) + 1.0)
    h1 = jnp.maximum((agg_ref[...] + hs_ref[...]) * dinv[:, None] + b_ref[...], 0.0)
    out_ref[...] = jnp.dot(h1, w_ref[...],
                           preferred_element_type=jnp.float32) * dinv[:, None]


def _mid_call(agg1, hs1, degp, W2, b1):
    return pl.pallas_call(
        _mid_body,
        grid=(NBLK,),
        in_specs=[
            pl.BlockSpec((RB, D), lambda i: (i, 0)),
            pl.BlockSpec((RB, D), lambda i: (i, 0)),
            pl.BlockSpec((RB, NC * NS), lambda i: (i, 0)),
            pl.BlockSpec((D, D), lambda i: (0, 0)),
            pl.BlockSpec((D,), lambda i: (0,)),
        ],
        out_specs=pl.BlockSpec((RB, D), lambda i: (i, 0)),
        out_shape=jax.ShapeDtypeStruct((N, D), jnp.float32),
    )(agg1, hs1, degp, W2, b1)


def _final_body(agg_ref, hs_ref, degp_ref, b_ref, batch_ref, glob_ref,
                wh1_ref, bh1_ref, wh2_ref, bh2_ref, out_ref,
                pooled_acc, cnt_acc):
    i = pl.program_id(0)
    dinv = lax.rsqrt(jnp.sum(degp_ref[...], axis=1) + 1.0)
    h2 = (agg_ref[...] + hs_ref[...]) * dinv[:, None] + b_ref[...]
    gids = lax.broadcasted_iota(jnp.int32, (RB, NG), 1)
    m = (gids == batch_ref[...]).astype(jnp.float32)

    @pl.when(i == 0)
    def _():
        pooled_acc[...] = jnp.zeros_like(pooled_acc)
        cnt_acc[...] = jnp.zeros_like(cnt_acc)

    pooled_acc[...] += lax.dot_general(m, h2, (((0,), (0,)), ((), ())),
                                       preferred_element_type=jnp.float32)
    cnt_acc[...] += jnp.broadcast_to(jnp.sum(m, axis=0)[:, None],
                                     cnt_acc.shape)

    @pl.when(i == NBLK - 1)
    def _():
        cnt = cnt_acc[...][:, 0:1]
        pooled = pooled_acc[...] / jnp.maximum(cnt, 1.0)
        z1 = (jnp.dot(pooled, wh1_ref[0:D, :], preferred_element_type=jnp.float32)
              + jnp.dot(glob_ref[...], wh1_ref[D:D + 10, :],
                        preferred_element_type=jnp.float32)
              + bh1_ref[...])
        z1 = jnp.maximum(z1, 0.0)
        z2 = jnp.dot(z1, wh2_ref[...], preferred_element_type=jnp.float32) + bh2_ref[...]
        out_ref[...] = jax.nn.sigmoid(z2)


def _final_call(agg2, hs2, degp, b2, batch2d, glob_vecs, Wh1, bh1, Wh2, bh2):
    return pl.pallas_call(
        _final_body,
        grid=(NBLK,),
        in_specs=[
            pl.BlockSpec((RB, D), lambda i: (i, 0)),
            pl.BlockSpec((RB, D), lambda i: (i, 0)),
            pl.BlockSpec((RB, NC * NS), lambda i: (i, 0)),
            pl.BlockSpec((D,), lambda i: (0,)),
            pl.BlockSpec((RB, 1), lambda i: (i, 0)),
            pl.BlockSpec((NG, 10), lambda i: (0, 0)),
            pl.BlockSpec((D + 10, 10), lambda i: (0, 0)),
            pl.BlockSpec((1, 10), lambda i: (0, 0)),
            pl.BlockSpec((10, 1), lambda i: (0, 0)),
            pl.BlockSpec((1, 1), lambda i: (0, 0)),
        ],
        out_specs=pl.BlockSpec((NG, 1), lambda i: (0, 0)),
        out_shape=jax.ShapeDtypeStruct((NG, 1), jnp.float32),
        scratch_shapes=[
            pltpu.VMEM((NG, D), jnp.float32),
            pltpu.VMEM((NG, 128), jnp.float32),
        ],
    )(agg2, hs2, degp, b2, batch2d, glob_vecs, Wh1, bh1, Wh2, bh2)


def kernel(x, edge_index, edge_attr, batch_idx, glob_vecs,
           W1, b1, W2, b2, Wh1, bh1, Wh2, bh2):
    pad = EPALLOC - E
    srcp = jnp.concatenate([edge_index[0], jnp.zeros((pad,), jnp.int32)])
    dstp = jnp.concatenate([edge_index[1], jnp.full((pad,), N, jnp.int32)])

    degp = _deg_call(dstp)[:, :N].T
    hs1 = _prep_call(x, W1, degp)
    agg1 = _agg_call(hs1.reshape(N * D), srcp, dstp)
    hs2 = _mid_call(agg1, hs1, degp, W2, b1)
    agg2 = _agg_call(hs2.reshape(N * D), srcp, dstp)
    z = _final_call(agg2, hs2, degp, b2, batch_idx.reshape(N, 1),
                    glob_vecs, Wh1, bh1.reshape(1, 10), Wh2, bh2.reshape(1, 1))
    return z
